# Initial kernel scaffold; baseline (speedup 1.0000x reference)
import jax
import jax.numpy as jnp
from jax.experimental import pallas as pl


def kernel(images, world_points, world_points_conf, extrinsic, intrinsic):
    raise NotImplementedError("write your pallas kernel here")


# speedup vs baseline: 163.4299x; 163.4299x over previous
import jax
import jax.numpy as jnp
from jax import lax
from jax.experimental import pallas as pl
from jax.experimental.pallas import tpu as pltpu
from jax.experimental.pallas import tpu_sc as plsc

_KF, _CH, _H, _W, _S = 4, 3, 96, 96, 32
_HW = _H * _W
_NPT = _KF * _H * _W
_CHUNK = 4608
_NCHUNK = _NPT // _CHUNK
_VPC = _CHUNK // 16
_NC = 2


def _tile_body(xs, ys, zs, cfh, rh, gh, bh, par,
               rgb_o, dep_o, con_o, msk_o, chk_o,
               xb, yb, zb, cb, rb, gb, bb,
               depth, ra, ga, ba, ca, mb, pb):
    wid = lax.axis_index("s") * _NC + lax.axis_index("c")
    s = wid
    pltpu.sync_copy(par.at[pl.ds(s * 48, 48)], pb)
    p_lo = pb[pl.ds(0, 16)]
    p_mid = pb[pl.ds(16, 16)]
    p_z = pb[pl.ds(32, 16)]
    e = [p_lo[i] for i in range(12)]
    km = [p_mid[i] for i in range(9)]
    zeri = plsc.bitcast(p_z, jnp.int32)

    def bar(vv):
        return plsc.bitcast(plsc.bitcast(vv, jnp.int32) ^ zeri,
                            jnp.float32)

    inf16 = jnp.full((16,), jnp.inf, dtype=jnp.float32)
    zero16 = jnp.zeros((16,), dtype=jnp.float32)

    def init_body(i, carry):
        sl = pl.ds(i * 16, 16)
        depth[sl] = inf16
        ra[sl] = zero16
        ga[sl] = zero16
        ba[sl] = zero16
        ca[sl] = zero16
        return carry
    lax.fori_loop(0, _HW // 16, init_body, 0)

    def project(x, y, z):
        t = [e[0] * x, e[1] * y, e[2] * z,
             e[4] * x, e[5] * y, e[6] * z,
             e[8] * x, e[9] * y, e[10] * z]
        c0 = bar(t[0] + t[1]) + bar(t[2] + e[3])
        c1 = bar(t[3] + t[4]) + bar(t[5] + e[7])
        c2 = bar(t[6] + t[7]) + bar(t[8] + e[11])
        k = [km[0] * c0, km[1] * c1, km[2] * c2,
             km[3] * c0, km[4] * c1, km[5] * c2,
             km[6] * c0, km[7] * c1, km[8] * c2]
        h0 = bar(k[0] + k[1]) + k[2]
        h1 = bar(k[3] + k[4]) + k[5]
        h2 = bar(k[6] + k[7]) + k[8]
        den = bar(h2) + 1e-7
        terms = t + k
        while len(terms) > 1:
            terms = [terms[i] + terms[i + 1]
                     for i in range(0, len(terms) - 1, 2)] \
                + ([terms[-1]] if len(terms) % 2 else [])
        return c2, h0 / den, h1 / den, terms[0]

    def chunk_a(c, carry):
        off = c * _CHUNK
        pltpu.sync_copy(xs.at[pl.ds(off, _CHUNK)], xb)
        pltpu.sync_copy(ys.at[pl.ds(off, _CHUNK)], yb)
        pltpu.sync_copy(zs.at[pl.ds(off, _CHUNK)], zb)

        iota16 = lax.iota(jnp.int32, 16)

        def vec_a(j, inner):
            sl = pl.ds(j * 16, 16)
            zf, u, v, chk = project(xb[sl], yb[sl], zb[sl])
            ucl = jnp.minimum(jnp.maximum(u, 0.0), 95.0)
            vcl = jnp.minimum(jnp.maximum(v, 0.0), 95.0)
            pix = vcl.astype(jnp.int32) * _W + ucl.astype(jnp.int32)
            sk, sv = plsc.sort_key_val(pix, zf)
            for dstep in (1, 2, 4, 8):
                idxm = jnp.maximum(iota16 - dstep, 0)
                psh = sk.at[idxm].get(mode="promise_in_bounds")
                zsh = sv.at[idxm].get(mode="promise_in_bounds")
                sv = jnp.where(psh == sk, jnp.minimum(sv, zsh), sv)
            nxt = sk.at[jnp.minimum(iota16 + 1, 15)].get(
                mode="promise_in_bounds")
            islast = (sk != nxt) | (iota16 == 15)
            cur = plsc.load_gather(depth, [sk])
            plsc.store_scatter(depth, [sk], jnp.minimum(cur, sv), mask=islast)
            return inner + chk
        return lax.fori_loop(0, _VPC, vec_a, carry)
    chkv = lax.fori_loop(0, _NCHUNK, chunk_a,
                         jnp.zeros((16,), jnp.float32))

    def chunk_b(c, carry):
        off = c * _CHUNK
        pltpu.sync_copy(xs.at[pl.ds(off, _CHUNK)], xb)
        pltpu.sync_copy(ys.at[pl.ds(off, _CHUNK)], yb)
        pltpu.sync_copy(zs.at[pl.ds(off, _CHUNK)], zb)
        pltpu.sync_copy(cfh.at[pl.ds(off, _CHUNK)], cb)
        pltpu.sync_copy(rh.at[pl.ds(off, _CHUNK)], rb)
        pltpu.sync_copy(gh.at[pl.ds(off, _CHUNK)], gb)
        pltpu.sync_copy(bh.at[pl.ds(off, _CHUNK)], bb)

        def vec_b(j, inner):
            sl = pl.ds(j * 16, 16)
            cf = cb[sl]
            zf, u, v, chk = project(xb[sl], yb[sl], zb[sl])
            valid = ((zf > 0.0) & (u >= 0.0) & (u < 96.0)
                     & (v >= 0.0) & (v < 96.0) & (cf >= 0.5))
            ucl = jnp.minimum(jnp.maximum(u, 0.0), 95.0)
            vcl = jnp.minimum(jnp.maximum(v, 0.0), 95.0)
            ui = ucl.astype(jnp.int32)
            vi = vcl.astype(jnp.int32)
            pix = vi * _W + ui
            d = plsc.load_gather(depth, [pix])
            hit = (zf == d) & valid
            nhit = jnp.sum(hit.astype(jnp.int32))

            def splat():
                rv = rb[sl]
                gv = gb[sl]
                bv = bb[sl]
                du = ucl - ui.astype(jnp.float32)
                dv = vcl - vi.astype(jnp.float32)
                uinc = jnp.where(ui < _W - 1, 1, 0)
                vinc = jnp.where(vi < _H - 1, _W, 0)
                w00 = (1.0 - du) * (1.0 - dv)
                w01 = (1.0 - du) * dv
                w10 = du * (1.0 - dv)
                w11 = du * dv
                corners = ((w00, pix), (w01, pix + vinc),
                           (w10, pix + uinc), (w11, pix + uinc + vinc))
                for w_, p_ in corners:
                    plsc.addupdate_scatter(ra, [p_], w_ * rv, mask=hit)
                    plsc.addupdate_scatter(ga, [p_], w_ * gv, mask=hit)
                    plsc.addupdate_scatter(ba, [p_], w_ * bv, mask=hit)
                    plsc.addupdate_scatter(ca, [p_], w_ * cf, mask=hit)

            lax.cond(nhit > 0, splat, lambda: None)
            return inner + chk
        return lax.fori_loop(0, _VPC, vec_b, carry)
    chkv = lax.fori_loop(0, _NCHUNK, chunk_b, chkv)

    def fin(i, carry):
        sl = pl.ds(i * 16, 16)
        r_ = ra[sl]
        g_ = ga[sl]
        b_ = ba[sl]
        d_ = depth[sl]
        over = (r_ > 1.0) | (g_ > 1.0) | (b_ > 1.0)
        ra[sl] = jnp.minimum(jnp.maximum(r_, 0.0), 1.0)
        ga[sl] = jnp.minimum(jnp.maximum(g_, 0.0), 1.0)
        ba[sl] = jnp.minimum(jnp.maximum(b_, 0.0), 1.0)
        mb[sl] = jnp.where((d_ < jnp.inf) & jnp.logical_not(over), 1.0, 0.0)
        return carry
    lax.fori_loop(0, _HW // 16, fin, 0)

    pltpu.sync_copy(ra, rgb_o.at[pl.ds((s * 3 + 0) * _HW, _HW)])
    pltpu.sync_copy(ga, rgb_o.at[pl.ds((s * 3 + 1) * _HW, _HW)])
    pltpu.sync_copy(ba, rgb_o.at[pl.ds((s * 3 + 2) * _HW, _HW)])
    pltpu.sync_copy(depth, dep_o.at[pl.ds(s * _HW, _HW)])
    pltpu.sync_copy(ca, con_o.at[pl.ds(s * _HW, _HW)])
    pltpu.sync_copy(mb, msk_o.at[pl.ds(s * _HW, _HW)])
    pb[pl.ds(0, 16)] = chkv
    pltpu.sync_copy(pb.at[pl.ds(0, 16)], chk_o.at[pl.ds(s * 16, 16)])


def _make_kernel(interpret=False):
    mesh = plsc.VectorSubcoreMesh(core_axis_name="c", subcore_axis_name="s",
                                  num_cores=_NC, num_subcores=16)
    f32 = jnp.float32
    return pl.kernel(
        _tile_body,
        out_type=(
            jax.ShapeDtypeStruct((_S * _CH * _HW,), f32),
            jax.ShapeDtypeStruct((_S * _HW,), f32),
            jax.ShapeDtypeStruct((_S * _HW,), f32),
            jax.ShapeDtypeStruct((_S * _HW,), f32),
            jax.ShapeDtypeStruct((_S * 16,), f32),
        ),
        mesh=mesh,
        scratch_types=(
            [pltpu.VMEM((_CHUNK,), f32) for _ in range(7)]
            + [pltpu.VMEM((_HW,), f32) for _ in range(6)]
            + [pltpu.VMEM((48,), f32)]
        ),
        compiler_params=pltpu.CompilerParams(needs_layout_passes=False),
        interpret=interpret,
    )


def kernel(images, world_points, world_points_conf, extrinsic, intrinsic):
    f32 = jnp.float32
    wp = world_points.reshape(-1, 3)
    xs = wp[:, 0].astype(f32)
    ys = wp[:, 1].astype(f32)
    zs = wp[:, 2].astype(f32)
    cf = world_points_conf.reshape(-1).astype(f32)
    rh = images[:, 0, :, :].reshape(-1).astype(f32)
    gh = images[:, 1, :, :].reshape(-1).astype(f32)
    bh = images[:, 2, :, :].reshape(-1).astype(f32)
    par = jnp.concatenate(
        [extrinsic.reshape(_S, 12).astype(f32),
         jnp.zeros((_S, 4), f32),
         intrinsic.reshape(_S, 9).astype(f32),
         jnp.zeros((_S, 23), f32)], axis=1).reshape(-1)
    rgb_o, dep_o, con_o, msk_o, _chk = _make_kernel()(
        xs, ys, zs, cf, rh, gh, bh, par)
    return (rgb_o.reshape(_S, _CH, _H, _W),
            dep_o.reshape(_S, 1, _H, _W),
            con_o.reshape(_S, 1, _H, _W),
            msk_o.reshape(_S, _H, _W, 1))
